# baseline (speedup 1.0000x reference)
import jax
import jax.numpy as jnp
from jax import lax
from jax.experimental import pallas as pl
from jax.experimental.pallas import tpu as pltpu
from jax.experimental.pallas import tpu_sc as plsc

N = 10000
NPAD = 10240
E = 320000
NS = 16
NC = 2
NW = NC * NS
CH = 128
W = 128
IB = 8
NCH_A = 80
NCH_B = 160
RPT = NPAD // NS
RBLK = 512
GRID = NPAD // RBLK


def _make_sc_sum(nch0, nch1):
  nchm = max(nch0, nch1)
  mesh = plsc.VectorSubcoreMesh(core_axis_name="c", subcore_axis_name="s")
  out_type = [jax.ShapeDtypeStruct((NC * NPAD, W), jnp.float32)]
  scratch = [
      pltpu.VMEM((2, IB, CH), jnp.int32),
      pltpu.VMEM((2, IB, CH), jnp.int32),
      pltpu.VMEM((2, CH, W), jnp.float32),
      pltpu.VMEM((16, W), jnp.float32),
      pltpu.VMEM_SHARED((NPAD, W), jnp.float32),
      pltpu.SemaphoreType.DMA,
      pltpu.SemaphoreType.DMA,
  ]
  def body(table, src_i, dst_i, sums, src_v, dst_v, gbufs, zbuf, acc,
           gsem, isem):
    cid = lax.axis_index("c")
    sid = lax.axis_index("s")
    wid = cid * NS + sid
    nblk = jnp.where(cid == 0, nch0 // IB, nch1 // IB)
    z16 = jnp.zeros((16,), jnp.float32)
    for r in range(16):
      for k in range(W // 16):
        zbuf[r, pl.ds(k * 16, 16)] = z16

    def zero_acc(i, c):
      pltpu.sync_copy(zbuf, acc.at[pl.ds(sid * RPT + i * 16, 16)])
      return c
    lax.fori_loop(0, RPT // 16, zero_acc, 0)

    plsc.subcore_barrier()

    pltpu.sync_copy(src_i.at[wid, pl.ds(0, IB)], src_v.at[0])
    pltpu.sync_copy(dst_i.at[wid, pl.ds(0, IB)], dst_v.at[0])
    pltpu.async_copy(table.at[src_v.at[0, 0]], gbufs.at[0], gsem)

    def block(o, c):
      p = lax.rem(o, 2)
      q = lax.rem(o + 1, 2)
      not_last = o < nblk - 1

      @pl.when(not_last)
      def _():
        pltpu.async_copy(src_i.at[wid, pl.ds((o + 1) * IB, IB)],
                         src_v.at[q], isem)
        pltpu.async_copy(dst_i.at[wid, pl.ds((o + 1) * IB, IB)],
                         dst_v.at[q], isem)
      for jj in range(IB):
        if jj + 1 < IB:
          pltpu.async_copy(table.at[src_v.at[p, jj + 1]],
                           gbufs.at[(jj + 1) % 2], gsem)
        else:
          @pl.when(not_last)
          def _():
            pltpu.make_async_copy(src_i.at[wid, pl.ds(0, IB)],
                                  src_v.at[q], isem).wait()
            pltpu.make_async_copy(dst_i.at[wid, pl.ds(0, IB)],
                                  dst_v.at[q], isem).wait()
            pltpu.async_copy(table.at[src_v.at[q, 0]], gbufs.at[0], gsem)
        pltpu.make_async_copy(table.at[src_v.at[p, jj]],
                              gbufs.at[jj % 2], gsem).wait()
        pltpu.sync_copy(gbufs.at[jj % 2], acc.at[dst_v.at[p, jj]], add=True)
      return c
    lax.fori_loop(0, nblk, block, 0)

    plsc.subcore_barrier()

    def wb(i, c):
      pltpu.sync_copy(acc.at[pl.ds(sid * RPT + i * CH, CH)], gbufs.at[0])
      pltpu.sync_copy(gbufs.at[0],
                      sums.at[pl.ds(cid * NPAD + sid * RPT + i * CH, CH)])
      return c
    lax.fori_loop(0, RPT // CH, wb, 0)

  return pl.kernel(body, out_type=tuple(out_type), mesh=mesh,
                   scratch_types=scratch)


def _make_sc_gather_count(nch):
  nblk = nch // IB
  mesh = plsc.VectorSubcoreMesh(core_axis_name="c", subcore_axis_name="s")
  out_type = jax.ShapeDtypeStruct((NC * NPAD, W), jnp.float32)
  scratch = [
      pltpu.VMEM((2, IB, CH), jnp.int32),
      pltpu.VMEM((2, IB, CH), jnp.int32),
      pltpu.VMEM((2, CH, W), jnp.float32),
      pltpu.VMEM((16, W), jnp.float32),
      pltpu.VMEM_SHARED((NPAD, W), jnp.float32),
      pltpu.SemaphoreType.DMA,
      pltpu.SemaphoreType.DMA,
  ]

  def body(table, src_i, dst_i, out, src_v, dst_v, gbufs, zbuf, acc,
           gsem, isem):
    cid = lax.axis_index("c")
    sid = lax.axis_index("s")
    wid = cid * NS + sid
    z16 = jnp.zeros((16,), jnp.float32)
    ones16 = jnp.full((16,), 1.0, jnp.float32)
    for r in range(16):
      for k in range(W // 16):
        zbuf[r, pl.ds(k * 16, 16)] = z16

    def zero_acc(i, c):
      pltpu.sync_copy(zbuf, acc.at[pl.ds(sid * RPT + i * 16, 16)])
      return c
    lax.fori_loop(0, RPT // 16, zero_acc, 0)

    @pl.when(cid == 1)
    def _():
      for r in range(CH):
        for k in range(W // 16):
          gbufs[0, r, pl.ds(k * 16, 16)] = ones16

    plsc.subcore_barrier()

    @pl.when(cid == 0)
    def _():
      pltpu.sync_copy(src_i.at[wid, pl.ds(0, IB)], src_v.at[0])
      pltpu.sync_copy(dst_i.at[wid, pl.ds(0, IB)], dst_v.at[0])
      pltpu.async_copy(table.at[src_v.at[0, 0]], gbufs.at[0], gsem)

      def block(o, c):
        p = lax.rem(o, 2)
        q = lax.rem(o + 1, 2)
        not_last = o < nblk - 1

        @pl.when(not_last)
        def _():
          pltpu.async_copy(src_i.at[wid, pl.ds((o + 1) * IB, IB)],
                           src_v.at[q], isem)
          pltpu.async_copy(dst_i.at[wid, pl.ds((o + 1) * IB, IB)],
                           dst_v.at[q], isem)
        for jj in range(IB):
          if jj + 1 < IB:
            pltpu.async_copy(table.at[src_v.at[p, jj + 1]],
                             gbufs.at[(jj + 1) % 2], gsem)
          else:
            @pl.when(not_last)
            def _():
              pltpu.make_async_copy(src_i.at[wid, pl.ds(0, IB)],
                                    src_v.at[q], isem).wait()
              pltpu.make_async_copy(dst_i.at[wid, pl.ds(0, IB)],
                                    dst_v.at[q], isem).wait()
              pltpu.async_copy(table.at[src_v.at[q, 0]], gbufs.at[0], gsem)
          pltpu.make_async_copy(table.at[src_v.at[p, jj]],
                                gbufs.at[jj % 2], gsem).wait()
          pltpu.sync_copy(gbufs.at[jj % 2], acc.at[dst_v.at[p, jj]], add=True)
        return c
      lax.fori_loop(0, nblk, block, 0)

    @pl.when(cid == 1)
    def _():
      def cstep(o, c):
        pltpu.sync_copy(dst_i.at[wid, pl.ds(o * IB, IB)], dst_v.at[0])
        for jj in range(IB):
          pltpu.sync_copy(gbufs.at[0], acc.at[dst_v.at[0, jj]], add=True)
        return c
      lax.fori_loop(0, nblk, cstep, 0)

    plsc.subcore_barrier()

    def wb(i, c):
      pltpu.sync_copy(acc.at[pl.ds(sid * RPT + i * CH, CH)], gbufs.at[1])
      pltpu.sync_copy(gbufs.at[1],
                      out.at[pl.ds(cid * NPAD + sid * RPT + i * CH, CH)])
      return c
    lax.fori_loop(0, RPT // CH, wb, 0)

  return pl.kernel(body, out_type=out_type, mesh=mesh, scratch_types=scratch)


def _make_sc_count(nch):
  mesh = plsc.VectorSubcoreMesh(core_axis_name="c", subcore_axis_name="s")
  out_type = jax.ShapeDtypeStruct((NC * NPAD, W), jnp.float32)
  scratch = [
      pltpu.VMEM((IB, CH), jnp.int32),
      pltpu.VMEM((CH, W), jnp.float32),
      pltpu.VMEM((16, W), jnp.float32),
      pltpu.VMEM_SHARED((NPAD, W), jnp.float32),
  ]

  def body(dst_i, cnt, dst_v, obuf, zbuf, acc):
    cid = lax.axis_index("c")
    sid = lax.axis_index("s")
    wid = cid * NS + sid
    z16 = jnp.zeros((16,), jnp.float32)
    ones16 = jnp.full((16,), 1.0, jnp.float32)
    for r in range(16):
      for k in range(W // 16):
        zbuf[r, pl.ds(k * 16, 16)] = z16

    def zero_acc(i, c):
      pltpu.sync_copy(zbuf, acc.at[pl.ds(sid * RPT + i * 16, 16)])
      return c
    lax.fori_loop(0, RPT // 16, zero_acc, 0)

    for r in range(CH):
      for k in range(W // 16):
        obuf[r, pl.ds(k * 16, 16)] = ones16

    plsc.subcore_barrier()

    def step(o, c):
      pltpu.sync_copy(dst_i.at[wid, pl.ds(o * IB, IB)], dst_v)
      for jj in range(IB):
        pltpu.sync_copy(obuf, acc.at[dst_v.at[jj]], add=True)
      return c
    lax.fori_loop(0, nch // IB, step, 0)

    plsc.subcore_barrier()

    def wb(i, c):
      pltpu.sync_copy(acc.at[pl.ds(sid * RPT + i * CH, CH)], obuf)
      pltpu.sync_copy(obuf, cnt.at[pl.ds(cid * NPAD + sid * RPT + i * CH, CH)])
      return c
    lax.fori_loop(0, RPT // CH, wb, 0)

  return pl.kernel(body, out_type=out_type, mesh=mesh, scratch_types=scratch)


def _bdot(a, b):
  return jnp.dot(a.astype(jnp.bfloat16), b.astype(jnp.bfloat16),
                 preferred_element_type=jnp.float32)


HB = 2560
HG = E // HB


def _hist_body(hi_r, lo_r, o_r):
  i = pl.program_id(0)

  @pl.when(i == 0)
  def _():
    o_r[...] = jnp.zeros_like(o_r)
  hi = hi_r[0]
  lo = lo_r[...]
  hit = (lax.broadcasted_iota(jnp.int32, (NPAD // 128, 1), 0).astype(
      jnp.float32) == hi).astype(jnp.bfloat16)
  lot = (lo == lax.broadcasted_iota(jnp.int32, (1, 128), 1).astype(
      jnp.float32)).astype(jnp.bfloat16)
  o_r[...] += jnp.dot(hit, lot, preferred_element_type=jnp.float32)


def _enc_body(x_r, w1_r, b1_r, g_r, bb_r, w2_r, b2_r, o_r):
  h = _bdot(x_r[...], w1_r[...]) + b1_r[...]
  m = jnp.mean(h, axis=-1, keepdims=True)
  c = h - m
  v = jnp.mean(c * c, axis=-1, keepdims=True)
  h = c * lax.rsqrt(v + 1e-5) * g_r[...] + bb_r[...]
  h = jnp.maximum(h, 0.0)
  o_r[...] = _bdot(h, w2_r[...]) + b2_r[...]


def _l1_body(s_r, cnt_r, h0_r, wl_r, bl_r, wr_r, a_r, o_r):
  inv = 1.0 / jnp.maximum(cnt_r[...], 1.0)
  agg = (s_r[0] + s_r[1]) * inv
  t = _bdot(agg, wl_r[...]) + _bdot(h0_r[...], wr_r[...]) + bl_r[...]
  h1 = jnp.where(t >= 0, t, a_r[...] * t)
  o_r[0] = h1[:, :128]
  o_r[1] = h1[:, 128:]


def _l2_body(s_r, cnt_r, h1_r, wl_r, bl_r, wr_r, a_r, wl3_r, wr3_r, y_r, z_r):
  inv = 1.0 / jnp.maximum(cnt_r[...], 1.0)
  t = (_bdot(s_r[0] * inv, wl_r[0:128])
       + _bdot(s_r[1] * inv, wl_r[128:256])
       + _bdot(h1_r[0], wr_r[0:128])
       + _bdot(h1_r[1], wr_r[128:256])
       + bl_r[...])
  h2 = jnp.where(t >= 0, t, a_r[...] * t)
  h2b = h2.astype(jnp.bfloat16)
  y_r[...] = jnp.dot(h2b, wl3_r[...].astype(jnp.bfloat16),
                     preferred_element_type=jnp.float32)
  z_r[...] = jnp.dot(h2b, wr3_r[...].astype(jnp.bfloat16),
                     preferred_element_type=jnp.float32)


def _fin_body(s_r, cnt_r, z_r, bl_r, a_r, o_r):
  inv = 1.0 / jnp.maximum(cnt_r[...], 1.0)
  t = (s_r[0] + s_r[1]) * inv + z_r[...] + bl_r[...]
  o_r[...] = jnp.where(t >= 0, t, a_r[...] * t)


def _full(shape):
  return pl.BlockSpec(shape, lambda i: (0,) * len(shape))


def _pad_src(v, tot):
  pad = jnp.arange(tot - E, dtype=jnp.int32) % N
  return jnp.concatenate([v, pad])


def _pad_dst(v, tot):
  return jnp.concatenate([v, jnp.full((tot - E,), NPAD - 1, jnp.int32)])


def _pad_edges_split(v, fill, nch0, nch1):
  nchm = max(nch0, nch1)
  cap0 = NS * nch0 * CH
  s0, s1 = v[:cap0], v[cap0:]
  a0 = s0.reshape(NS, nch0, CH)
  a0 = jnp.concatenate(
      [a0, jnp.full((NS, nchm - nch0, CH), fill, jnp.int32)], axis=1)
  pad1 = NS * nch1 * CH - (E - cap0)
  a1 = jnp.concatenate([s1, jnp.full((pad1,), fill, jnp.int32)])
  a1 = a1.reshape(NS, nch1, CH)
  a1 = jnp.concatenate(
      [a1, jnp.full((NS, nchm - nch1, CH), fill, jnp.int32)], axis=1)
  return jnp.concatenate([a0, a1], axis=0)


def kernel(x, edge_index, batch_size, enc_w1, enc_b1, ln_g, ln_b, enc_w2,
           enc_b2, wl1, bl1, wr1, wl2, bl2, wr2, wl3, bl3, wr3, a1, a2, a3):
  f32 = jnp.float32
  xpad = jnp.zeros((NPAD, x.shape[1]), f32).at[:N].set(x)
  src = edge_index[0].astype(jnp.int32)
  dst = edge_index[1].astype(jnp.int32)

  src_a = _pad_src(src, NW * NCH_A * CH).reshape(NW, NCH_A, CH)
  dst_a = _pad_dst(dst, NW * NCH_A * CH).reshape(NW, NCH_A, CH)
  src_h = _pad_src(src, NS * NCH_B * CH).reshape(NS, NCH_B, CH)
  src_b = jnp.concatenate([src_h, src_h + NPAD], axis=0)
  dst_h = _pad_dst(dst, NS * NCH_B * CH).reshape(NS, NCH_B, CH)
  dst_b = jnp.concatenate([dst_h, dst_h], axis=0)

  enc = pl.pallas_call(
      _enc_body,
      grid=(GRID,),
      in_specs=[
          pl.BlockSpec((RBLK, 128), lambda i: (i, 0)),
          _full((128, 256)), _full((1, 256)), _full((1, 256)),
          _full((1, 256)), _full((256, 128)), _full((1, 128)),
      ],
      out_specs=pl.BlockSpec((RBLK, 128), lambda i: (i, 0)),
      out_shape=jax.ShapeDtypeStruct((NPAD, 128), f32),
  )
  h0 = enc(xpad, enc_w1, enc_b1.reshape(1, -1), ln_g.reshape(1, -1),
           ln_b.reshape(1, -1), enc_w2, enc_b2.reshape(1, -1))

  hist = pl.pallas_call(
      _hist_body,
      grid=(HG,),
      in_specs=[
          pl.BlockSpec((1, 1, HB), lambda i: (i, 0, 0)),
          pl.BlockSpec((HB, 1), lambda i: (i, 0)),
      ],
      out_specs=pl.BlockSpec((NPAD // 128, 128), lambda i: (0, 0)),
      out_shape=jax.ShapeDtypeStruct((NPAD // 128, 128), f32),
  )
  hi = (dst // 128).astype(f32).reshape(HG, 1, HB)
  lo = (dst % 128).astype(f32).reshape(E, 1)
  cntm = hist(hi, lo).reshape(NPAD, 1)

  (sums0,) = _make_sc_sum(NCH_A, NCH_A)(h0, src_a, dst_a)

  l1 = pl.pallas_call(
      _l1_body,
      grid=(GRID,),
      in_specs=[
          pl.BlockSpec((2, RBLK, 128), lambda i: (0, i, 0)),
          pl.BlockSpec((RBLK, 1), lambda i: (i, 0)),
          pl.BlockSpec((RBLK, 128), lambda i: (i, 0)),
          _full((128, 256)), _full((1, 256)), _full((128, 256)),
          _full((1, 256)),
      ],
      out_specs=pl.BlockSpec((2, RBLK, 128), lambda i: (0, i, 0)),
      out_shape=jax.ShapeDtypeStruct((2, NPAD, 128), f32),
  )
  h1c = l1(sums0.reshape(2, NPAD, 128), cntm, h0,
           wl1, bl1.reshape(1, -1), wr1, a1.reshape(1, -1))

  (sums1,) = _make_sc_sum(NCH_B, NCH_B)(h1c.reshape(NC * NPAD, 128), src_b, dst_b)

  l2 = pl.pallas_call(
      _l2_body,
      grid=(GRID,),
      in_specs=[
          pl.BlockSpec((2, RBLK, 128), lambda i: (0, i, 0)),
          pl.BlockSpec((RBLK, 1), lambda i: (i, 0)),
          pl.BlockSpec((2, RBLK, 128), lambda i: (0, i, 0)),
          _full((256, 512)), _full((1, 512)), _full((256, 512)),
          _full((1, 512)), _full((512, 128)), _full((512, 128)),
      ],
      out_specs=[
          pl.BlockSpec((RBLK, 128), lambda i: (i, 0)),
          pl.BlockSpec((RBLK, 128), lambda i: (i, 0)),
      ],
      out_shape=[
          jax.ShapeDtypeStruct((NPAD, 128), f32),
          jax.ShapeDtypeStruct((NPAD, 128), f32),
      ],
  )
  y, z = l2(sums1.reshape(2, NPAD, 128), cntm, h1c,
            wl2, bl2.reshape(1, -1), wr2, a2.reshape(1, -1), wl3, wr3)

  (sums2,) = _make_sc_sum(NCH_A, NCH_A)(y, src_a, dst_a)

  fin = pl.pallas_call(
      _fin_body,
      grid=(GRID,),
      in_specs=[
          pl.BlockSpec((2, RBLK, 128), lambda i: (0, i, 0)),
          pl.BlockSpec((RBLK, 1), lambda i: (i, 0)),
          pl.BlockSpec((RBLK, 128), lambda i: (i, 0)),
          _full((1, 128)), _full((1, 128)),
      ],
      out_specs=pl.BlockSpec((RBLK, 128), lambda i: (i, 0)),
      out_shape=jax.ShapeDtypeStruct((NPAD, 128), f32),
  )
  o = fin(sums2.reshape(2, NPAD, 128), cntm, z,
          bl3.reshape(1, -1), a3.reshape(1, -1))

  return lax.dynamic_slice_in_dim(o[:N], batch_size - 1024, 1024, axis=0)


# speedup vs baseline: 1.2673x; 1.2673x over previous
import jax
import jax.numpy as jnp
from jax import lax
from jax.experimental import pallas as pl
from jax.experimental.pallas import tpu as pltpu
from jax.experimental.pallas import tpu_sc as plsc

N = 10000
NPAD = 10240
E = 320000
NS = 16
NC = 2
NW = NC * NS
CH = 128
W = 128
IB = 8
NCH_A = 80
NCH_B = 160
RPT = NPAD // NS
RBLK = 512
GRID = NPAD // RBLK


def _make_sc_sum(nch0, nch1):
  nchm = max(nch0, nch1)
  mesh = plsc.VectorSubcoreMesh(core_axis_name="c", subcore_axis_name="s")
  out_type = [jax.ShapeDtypeStruct((NC * NPAD, W), jnp.float32)]
  scratch = [
      pltpu.VMEM((2, IB, CH), jnp.int32),
      pltpu.VMEM((2, IB, CH), jnp.int32),
      pltpu.VMEM((2, CH, W), jnp.float32),
      pltpu.VMEM((16, W), jnp.float32),
      pltpu.VMEM_SHARED((NPAD, W), jnp.float32),
      pltpu.SemaphoreType.DMA,
      pltpu.SemaphoreType.DMA,
  ]
  def body(table, src_i, dst_i, sums, src_v, dst_v, gbufs, zbuf, acc,
           gsem, isem):
    cid = lax.axis_index("c")
    sid = lax.axis_index("s")
    wid = cid * NS + sid
    nblk = jnp.where(cid == 0, nch0 // IB, nch1 // IB)
    z16 = jnp.zeros((16,), jnp.float32)
    for r in range(16):
      for k in range(W // 16):
        zbuf[r, pl.ds(k * 16, 16)] = z16

    def zero_acc(i, c):
      pltpu.sync_copy(zbuf, acc.at[pl.ds(sid * RPT + i * 16, 16)])
      return c
    lax.fori_loop(0, RPT // 16, zero_acc, 0)

    plsc.subcore_barrier()

    pltpu.sync_copy(src_i.at[wid, pl.ds(0, IB)], src_v.at[0])
    pltpu.sync_copy(dst_i.at[wid, pl.ds(0, IB)], dst_v.at[0])
    pltpu.async_copy(table.at[src_v.at[0, 0]], gbufs.at[0], gsem)

    def block(o, c):
      p = lax.rem(o, 2)
      q = lax.rem(o + 1, 2)
      not_last = o < nblk - 1

      @pl.when(not_last)
      def _():
        pltpu.async_copy(src_i.at[wid, pl.ds((o + 1) * IB, IB)],
                         src_v.at[q], isem)
        pltpu.async_copy(dst_i.at[wid, pl.ds((o + 1) * IB, IB)],
                         dst_v.at[q], isem)
      for jj in range(IB):
        if jj + 1 < IB:
          pltpu.async_copy(table.at[src_v.at[p, jj + 1]],
                           gbufs.at[(jj + 1) % 2], gsem)
        else:
          @pl.when(not_last)
          def _():
            pltpu.make_async_copy(src_i.at[wid, pl.ds(0, IB)],
                                  src_v.at[q], isem).wait()
            pltpu.make_async_copy(dst_i.at[wid, pl.ds(0, IB)],
                                  dst_v.at[q], isem).wait()
            pltpu.async_copy(table.at[src_v.at[q, 0]], gbufs.at[0], gsem)
        pltpu.make_async_copy(table.at[src_v.at[p, jj]],
                              gbufs.at[jj % 2], gsem).wait()
        pltpu.sync_copy(gbufs.at[jj % 2], acc.at[dst_v.at[p, jj]], add=True)
      return c
    lax.fori_loop(0, nblk, block, 0)

    plsc.subcore_barrier()

    def wb(i, c):
      pltpu.sync_copy(acc.at[pl.ds(sid * RPT + i * CH, CH)], gbufs.at[0])
      pltpu.sync_copy(gbufs.at[0],
                      sums.at[pl.ds(cid * NPAD + sid * RPT + i * CH, CH)])
      return c
    lax.fori_loop(0, RPT // CH, wb, 0)

  return pl.kernel(body, out_type=tuple(out_type), mesh=mesh,
                   scratch_types=scratch)


def _make_sc_gather_count(nch):
  nblk = nch // IB
  mesh = plsc.VectorSubcoreMesh(core_axis_name="c", subcore_axis_name="s")
  out_type = jax.ShapeDtypeStruct((NC * NPAD, W), jnp.float32)
  scratch = [
      pltpu.VMEM((2, IB, CH), jnp.int32),
      pltpu.VMEM((2, IB, CH), jnp.int32),
      pltpu.VMEM((2, CH, W), jnp.float32),
      pltpu.VMEM((16, W), jnp.float32),
      pltpu.VMEM_SHARED((NPAD, W), jnp.float32),
      pltpu.SemaphoreType.DMA,
      pltpu.SemaphoreType.DMA,
  ]

  def body(table, src_i, dst_i, out, src_v, dst_v, gbufs, zbuf, acc,
           gsem, isem):
    cid = lax.axis_index("c")
    sid = lax.axis_index("s")
    wid = cid * NS + sid
    z16 = jnp.zeros((16,), jnp.float32)
    ones16 = jnp.full((16,), 1.0, jnp.float32)
    for r in range(16):
      for k in range(W // 16):
        zbuf[r, pl.ds(k * 16, 16)] = z16

    def zero_acc(i, c):
      pltpu.sync_copy(zbuf, acc.at[pl.ds(sid * RPT + i * 16, 16)])
      return c
    lax.fori_loop(0, RPT // 16, zero_acc, 0)

    @pl.when(cid == 1)
    def _():
      for r in range(CH):
        for k in range(W // 16):
          gbufs[0, r, pl.ds(k * 16, 16)] = ones16

    plsc.subcore_barrier()

    @pl.when(cid == 0)
    def _():
      pltpu.sync_copy(src_i.at[wid, pl.ds(0, IB)], src_v.at[0])
      pltpu.sync_copy(dst_i.at[wid, pl.ds(0, IB)], dst_v.at[0])
      pltpu.async_copy(table.at[src_v.at[0, 0]], gbufs.at[0], gsem)

      def block(o, c):
        p = lax.rem(o, 2)
        q = lax.rem(o + 1, 2)
        not_last = o < nblk - 1

        @pl.when(not_last)
        def _():
          pltpu.async_copy(src_i.at[wid, pl.ds((o + 1) * IB, IB)],
                           src_v.at[q], isem)
          pltpu.async_copy(dst_i.at[wid, pl.ds((o + 1) * IB, IB)],
                           dst_v.at[q], isem)
        for jj in range(IB):
          if jj + 1 < IB:
            pltpu.async_copy(table.at[src_v.at[p, jj + 1]],
                             gbufs.at[(jj + 1) % 2], gsem)
          else:
            @pl.when(not_last)
            def _():
              pltpu.make_async_copy(src_i.at[wid, pl.ds(0, IB)],
                                    src_v.at[q], isem).wait()
              pltpu.make_async_copy(dst_i.at[wid, pl.ds(0, IB)],
                                    dst_v.at[q], isem).wait()
              pltpu.async_copy(table.at[src_v.at[q, 0]], gbufs.at[0], gsem)
          pltpu.make_async_copy(table.at[src_v.at[p, jj]],
                                gbufs.at[jj % 2], gsem).wait()
          pltpu.sync_copy(gbufs.at[jj % 2], acc.at[dst_v.at[p, jj]], add=True)
        return c
      lax.fori_loop(0, nblk, block, 0)

    @pl.when(cid == 1)
    def _():
      def cstep(o, c):
        pltpu.sync_copy(dst_i.at[wid, pl.ds(o * IB, IB)], dst_v.at[0])
        for jj in range(IB):
          pltpu.sync_copy(gbufs.at[0], acc.at[dst_v.at[0, jj]], add=True)
        return c
      lax.fori_loop(0, nblk, cstep, 0)

    plsc.subcore_barrier()

    def wb(i, c):
      pltpu.sync_copy(acc.at[pl.ds(sid * RPT + i * CH, CH)], gbufs.at[1])
      pltpu.sync_copy(gbufs.at[1],
                      out.at[pl.ds(cid * NPAD + sid * RPT + i * CH, CH)])
      return c
    lax.fori_loop(0, RPT // CH, wb, 0)

  return pl.kernel(body, out_type=out_type, mesh=mesh, scratch_types=scratch)


def _make_sc_count(nch):
  mesh = plsc.VectorSubcoreMesh(core_axis_name="c", subcore_axis_name="s")
  out_type = jax.ShapeDtypeStruct((NC * NPAD, W), jnp.float32)
  scratch = [
      pltpu.VMEM((IB, CH), jnp.int32),
      pltpu.VMEM((CH, W), jnp.float32),
      pltpu.VMEM((16, W), jnp.float32),
      pltpu.VMEM_SHARED((NPAD, W), jnp.float32),
  ]

  def body(dst_i, cnt, dst_v, obuf, zbuf, acc):
    cid = lax.axis_index("c")
    sid = lax.axis_index("s")
    wid = cid * NS + sid
    z16 = jnp.zeros((16,), jnp.float32)
    ones16 = jnp.full((16,), 1.0, jnp.float32)
    for r in range(16):
      for k in range(W // 16):
        zbuf[r, pl.ds(k * 16, 16)] = z16

    def zero_acc(i, c):
      pltpu.sync_copy(zbuf, acc.at[pl.ds(sid * RPT + i * 16, 16)])
      return c
    lax.fori_loop(0, RPT // 16, zero_acc, 0)

    for r in range(CH):
      for k in range(W // 16):
        obuf[r, pl.ds(k * 16, 16)] = ones16

    plsc.subcore_barrier()

    def step(o, c):
      pltpu.sync_copy(dst_i.at[wid, pl.ds(o * IB, IB)], dst_v)
      for jj in range(IB):
        pltpu.sync_copy(obuf, acc.at[dst_v.at[jj]], add=True)
      return c
    lax.fori_loop(0, nch // IB, step, 0)

    plsc.subcore_barrier()

    def wb(i, c):
      pltpu.sync_copy(acc.at[pl.ds(sid * RPT + i * CH, CH)], obuf)
      pltpu.sync_copy(obuf, cnt.at[pl.ds(cid * NPAD + sid * RPT + i * CH, CH)])
      return c
    lax.fori_loop(0, RPT // CH, wb, 0)

  return pl.kernel(body, out_type=out_type, mesh=mesh, scratch_types=scratch)


def _bdot(a, b):
  return jnp.dot(a.astype(jnp.bfloat16), b.astype(jnp.bfloat16),
                 preferred_element_type=jnp.float32)


def _enc_body(x_r, w1_r, b1_r, g_r, bb_r, w2_r, b2_r, o_r):
  h = _bdot(x_r[...], w1_r[...]) + b1_r[...]
  m = jnp.mean(h, axis=-1, keepdims=True)
  c = h - m
  v = jnp.mean(c * c, axis=-1, keepdims=True)
  h = c * lax.rsqrt(v + 1e-5) * g_r[...] + bb_r[...]
  h = jnp.maximum(h, 0.0)
  o_r[...] = _bdot(h, w2_r[...]) + b2_r[...]


def _l1_body(s_r, cnt_r, h0_r, wl_r, bl_r, wr_r, a_r, o_r):
  inv = 1.0 / jnp.maximum(cnt_r[0][:, 0:1] + cnt_r[1][:, 0:1], 1.0)
  agg = (s_r[0] + s_r[1]) * inv
  t = _bdot(agg, wl_r[...]) + _bdot(h0_r[...], wr_r[...]) + bl_r[...]
  h1 = jnp.where(t >= 0, t, a_r[...] * t)
  o_r[0] = h1[:, :128]
  o_r[1] = h1[:, 128:]


def _l2_body(s_r, cnt_r, h1_r, wl_r, bl_r, wr_r, a_r, wl3_r, wr3_r, y_r, z_r):
  inv = 1.0 / jnp.maximum(cnt_r[0][:, 0:1] + cnt_r[1][:, 0:1], 1.0)
  t = (_bdot(s_r[0] * inv, wl_r[0:128])
       + _bdot(s_r[1] * inv, wl_r[128:256])
       + _bdot(h1_r[0], wr_r[0:128])
       + _bdot(h1_r[1], wr_r[128:256])
       + bl_r[...])
  h2 = jnp.where(t >= 0, t, a_r[...] * t)
  h2b = h2.astype(jnp.bfloat16)
  y_r[...] = jnp.dot(h2b, wl3_r[...].astype(jnp.bfloat16),
                     preferred_element_type=jnp.float32)
  z_r[...] = jnp.dot(h2b, wr3_r[...].astype(jnp.bfloat16),
                     preferred_element_type=jnp.float32)


def _fin_body(s_r, cnt_r, z_r, bl_r, a_r, o_r):
  inv = 1.0 / jnp.maximum(cnt_r[0][:, 0:1] + cnt_r[1][:, 0:1], 1.0)
  t = (s_r[0] + s_r[1]) * inv + z_r[...] + bl_r[...]
  o_r[...] = jnp.where(t >= 0, t, a_r[...] * t)


def _full(shape):
  return pl.BlockSpec(shape, lambda i: (0,) * len(shape))


def _pad_src(v, tot):
  pad = jnp.arange(tot - E, dtype=jnp.int32) % N
  return jnp.concatenate([v, pad])


def _pad_dst(v, tot):
  return jnp.concatenate([v, jnp.full((tot - E,), NPAD - 1, jnp.int32)])


def _pad_edges_split(v, fill, nch0, nch1):
  nchm = max(nch0, nch1)
  cap0 = NS * nch0 * CH
  s0, s1 = v[:cap0], v[cap0:]
  a0 = s0.reshape(NS, nch0, CH)
  a0 = jnp.concatenate(
      [a0, jnp.full((NS, nchm - nch0, CH), fill, jnp.int32)], axis=1)
  pad1 = NS * nch1 * CH - (E - cap0)
  a1 = jnp.concatenate([s1, jnp.full((pad1,), fill, jnp.int32)])
  a1 = a1.reshape(NS, nch1, CH)
  a1 = jnp.concatenate(
      [a1, jnp.full((NS, nchm - nch1, CH), fill, jnp.int32)], axis=1)
  return jnp.concatenate([a0, a1], axis=0)


def kernel(x, edge_index, batch_size, enc_w1, enc_b1, ln_g, ln_b, enc_w2,
           enc_b2, wl1, bl1, wr1, wl2, bl2, wr2, wl3, bl3, wr3, a1, a2, a3):
  f32 = jnp.float32
  xpad = jnp.zeros((NPAD, x.shape[1]), f32).at[:N].set(x)
  src = edge_index[0].astype(jnp.int32)
  dst = edge_index[1].astype(jnp.int32)

  src_a = _pad_src(src, NW * NCH_A * CH).reshape(NW, NCH_A, CH)
  dst_a = _pad_dst(dst, NW * NCH_A * CH).reshape(NW, NCH_A, CH)
  src_h = _pad_src(src, NS * NCH_B * CH).reshape(NS, NCH_B, CH)
  src_b = jnp.concatenate([src_h, src_h + NPAD], axis=0)
  dst_h = _pad_dst(dst, NS * NCH_B * CH).reshape(NS, NCH_B, CH)
  dst_b = jnp.concatenate([dst_h, dst_h], axis=0)

  enc = pl.pallas_call(
      _enc_body,
      grid=(GRID,),
      in_specs=[
          pl.BlockSpec((RBLK, 128), lambda i: (i, 0)),
          _full((128, 256)), _full((1, 256)), _full((1, 256)),
          _full((1, 256)), _full((256, 128)), _full((1, 128)),
      ],
      out_specs=pl.BlockSpec((RBLK, 128), lambda i: (i, 0)),
      out_shape=jax.ShapeDtypeStruct((NPAD, 128), f32),
  )
  h0 = enc(xpad, enc_w1, enc_b1.reshape(1, -1), ln_g.reshape(1, -1),
           ln_b.reshape(1, -1), enc_w2, enc_b2.reshape(1, -1))

  (sums0,) = _make_sc_sum(NCH_A, NCH_A)(h0, src_a, dst_a)
  cntm = _make_sc_count(NCH_A)(dst_a).reshape(2, NPAD, 128)

  l1 = pl.pallas_call(
      _l1_body,
      grid=(GRID,),
      in_specs=[
          pl.BlockSpec((2, RBLK, 128), lambda i: (0, i, 0)),
          pl.BlockSpec((2, RBLK, 128), lambda i: (0, i, 0)),
          pl.BlockSpec((RBLK, 128), lambda i: (i, 0)),
          _full((128, 256)), _full((1, 256)), _full((128, 256)),
          _full((1, 256)),
      ],
      out_specs=pl.BlockSpec((2, RBLK, 128), lambda i: (0, i, 0)),
      out_shape=jax.ShapeDtypeStruct((2, NPAD, 128), f32),
  )
  h1c = l1(sums0.reshape(2, NPAD, 128), cntm, h0,
           wl1, bl1.reshape(1, -1), wr1, a1.reshape(1, -1))

  (sums1,) = _make_sc_sum(NCH_B, NCH_B)(h1c.reshape(NC * NPAD, 128), src_b, dst_b)

  l2 = pl.pallas_call(
      _l2_body,
      grid=(GRID,),
      in_specs=[
          pl.BlockSpec((2, RBLK, 128), lambda i: (0, i, 0)),
          pl.BlockSpec((2, RBLK, 128), lambda i: (0, i, 0)),
          pl.BlockSpec((2, RBLK, 128), lambda i: (0, i, 0)),
          _full((256, 512)), _full((1, 512)), _full((256, 512)),
          _full((1, 512)), _full((512, 128)), _full((512, 128)),
      ],
      out_specs=[
          pl.BlockSpec((RBLK, 128), lambda i: (i, 0)),
          pl.BlockSpec((RBLK, 128), lambda i: (i, 0)),
      ],
      out_shape=[
          jax.ShapeDtypeStruct((NPAD, 128), f32),
          jax.ShapeDtypeStruct((NPAD, 128), f32),
      ],
  )
  y, z = l2(sums1.reshape(2, NPAD, 128), cntm, h1c,
            wl2, bl2.reshape(1, -1), wr2, a2.reshape(1, -1), wl3, wr3)

  (sums2,) = _make_sc_sum(NCH_A, NCH_A)(y, src_a, dst_a)

  fin = pl.pallas_call(
      _fin_body,
      grid=(GRID,),
      in_specs=[
          pl.BlockSpec((2, RBLK, 128), lambda i: (0, i, 0)),
          pl.BlockSpec((2, RBLK, 128), lambda i: (0, i, 0)),
          pl.BlockSpec((RBLK, 128), lambda i: (i, 0)),
          _full((1, 128)), _full((1, 128)),
      ],
      out_specs=pl.BlockSpec((RBLK, 128), lambda i: (i, 0)),
      out_shape=jax.ShapeDtypeStruct((NPAD, 128), f32),
  )
  o = fin(sums2.reshape(2, NPAD, 128), cntm, z,
          bl3.reshape(1, -1), a3.reshape(1, -1))

  return lax.dynamic_slice_in_dim(o[:N], batch_size - 1024, 1024, axis=0)


# speedup vs baseline: 1.3297x; 1.0493x over previous
import jax
import jax.numpy as jnp
from jax import lax
from jax.experimental import pallas as pl
from jax.experimental.pallas import tpu as pltpu
from jax.experimental.pallas import tpu_sc as plsc

N = 10000
NPAD = 10240
E = 320000
NS = 16
NC = 2
NW = NC * NS
CH = 128
W = 128
IB = 8
NCH_A = 80
NCH_B = 160
RPT = NPAD // NS
RBLK = 1024
GRID = NPAD // RBLK


def _make_sc_sum(nch0, nch1):
  nchm = max(nch0, nch1)
  mesh = plsc.VectorSubcoreMesh(core_axis_name="c", subcore_axis_name="s")
  out_type = [jax.ShapeDtypeStruct((NC * NPAD, W), jnp.float32)]
  scratch = [
      pltpu.VMEM((2, IB, CH), jnp.int32),
      pltpu.VMEM((2, IB, CH), jnp.int32),
      pltpu.VMEM((2, CH, W), jnp.float32),
      pltpu.VMEM((16, W), jnp.float32),
      pltpu.VMEM_SHARED((NPAD, W), jnp.float32),
      pltpu.SemaphoreType.DMA,
      pltpu.SemaphoreType.DMA,
  ]
  def body(table, src_i, dst_i, sums, src_v, dst_v, gbufs, zbuf, acc,
           gsem, isem):
    cid = lax.axis_index("c")
    sid = lax.axis_index("s")
    wid = cid * NS + sid
    nblk = jnp.where(cid == 0, nch0 // IB, nch1 // IB)
    z16 = jnp.zeros((16,), jnp.float32)
    for r in range(16):
      for k in range(W // 16):
        zbuf[r, pl.ds(k * 16, 16)] = z16

    def zero_acc(i, c):
      pltpu.sync_copy(zbuf, acc.at[pl.ds(sid * RPT + i * 16, 16)])
      return c
    lax.fori_loop(0, RPT // 16, zero_acc, 0)

    plsc.subcore_barrier()

    pltpu.sync_copy(src_i.at[wid, pl.ds(0, IB)], src_v.at[0])
    pltpu.sync_copy(dst_i.at[wid, pl.ds(0, IB)], dst_v.at[0])
    pltpu.async_copy(table.at[src_v.at[0, 0]], gbufs.at[0], gsem)

    def block(o, c):
      p = lax.rem(o, 2)
      q = lax.rem(o + 1, 2)
      not_last = o < nblk - 1

      @pl.when(not_last)
      def _():
        pltpu.async_copy(src_i.at[wid, pl.ds((o + 1) * IB, IB)],
                         src_v.at[q], isem)
        pltpu.async_copy(dst_i.at[wid, pl.ds((o + 1) * IB, IB)],
                         dst_v.at[q], isem)
      for jj in range(IB):
        if jj + 1 < IB:
          pltpu.async_copy(table.at[src_v.at[p, jj + 1]],
                           gbufs.at[(jj + 1) % 2], gsem)
        else:
          @pl.when(not_last)
          def _():
            pltpu.make_async_copy(src_i.at[wid, pl.ds(0, IB)],
                                  src_v.at[q], isem).wait()
            pltpu.make_async_copy(dst_i.at[wid, pl.ds(0, IB)],
                                  dst_v.at[q], isem).wait()
            pltpu.async_copy(table.at[src_v.at[q, 0]], gbufs.at[0], gsem)
        pltpu.make_async_copy(table.at[src_v.at[p, jj]],
                              gbufs.at[jj % 2], gsem).wait()
        pltpu.sync_copy(gbufs.at[jj % 2], acc.at[dst_v.at[p, jj]], add=True)
      return c
    lax.fori_loop(0, nblk, block, 0)

    plsc.subcore_barrier()

    def wb(i, c):
      pltpu.sync_copy(acc.at[pl.ds(sid * RPT + i * CH, CH)], gbufs.at[0])
      pltpu.sync_copy(gbufs.at[0],
                      sums.at[pl.ds(cid * NPAD + sid * RPT + i * CH, CH)])
      return c
    lax.fori_loop(0, RPT // CH, wb, 0)

  return pl.kernel(body, out_type=tuple(out_type), mesh=mesh,
                   scratch_types=scratch)


def _make_sc_gather_count(nch):
  nblk = nch // IB
  mesh = plsc.VectorSubcoreMesh(core_axis_name="c", subcore_axis_name="s")
  out_type = jax.ShapeDtypeStruct((NC * NPAD, W), jnp.float32)
  scratch = [
      pltpu.VMEM((2, IB, CH), jnp.int32),
      pltpu.VMEM((2, IB, CH), jnp.int32),
      pltpu.VMEM((2, CH, W), jnp.float32),
      pltpu.VMEM((16, W), jnp.float32),
      pltpu.VMEM_SHARED((NPAD, W), jnp.float32),
      pltpu.SemaphoreType.DMA,
      pltpu.SemaphoreType.DMA,
  ]

  def body(table, src_i, dst_i, out, src_v, dst_v, gbufs, zbuf, acc,
           gsem, isem):
    cid = lax.axis_index("c")
    sid = lax.axis_index("s")
    wid = cid * NS + sid
    z16 = jnp.zeros((16,), jnp.float32)
    ones16 = jnp.full((16,), 1.0, jnp.float32)
    for r in range(16):
      for k in range(W // 16):
        zbuf[r, pl.ds(k * 16, 16)] = z16

    def zero_acc(i, c):
      pltpu.sync_copy(zbuf, acc.at[pl.ds(sid * RPT + i * 16, 16)])
      return c
    lax.fori_loop(0, RPT // 16, zero_acc, 0)

    @pl.when(cid == 1)
    def _():
      for r in range(CH):
        for k in range(W // 16):
          gbufs[0, r, pl.ds(k * 16, 16)] = ones16

    plsc.subcore_barrier()

    @pl.when(cid == 0)
    def _():
      pltpu.sync_copy(src_i.at[wid, pl.ds(0, IB)], src_v.at[0])
      pltpu.sync_copy(dst_i.at[wid, pl.ds(0, IB)], dst_v.at[0])
      pltpu.async_copy(table.at[src_v.at[0, 0]], gbufs.at[0], gsem)

      def block(o, c):
        p = lax.rem(o, 2)
        q = lax.rem(o + 1, 2)
        not_last = o < nblk - 1

        @pl.when(not_last)
        def _():
          pltpu.async_copy(src_i.at[wid, pl.ds((o + 1) * IB, IB)],
                           src_v.at[q], isem)
          pltpu.async_copy(dst_i.at[wid, pl.ds((o + 1) * IB, IB)],
                           dst_v.at[q], isem)
        for jj in range(IB):
          if jj + 1 < IB:
            pltpu.async_copy(table.at[src_v.at[p, jj + 1]],
                             gbufs.at[(jj + 1) % 2], gsem)
          else:
            @pl.when(not_last)
            def _():
              pltpu.make_async_copy(src_i.at[wid, pl.ds(0, IB)],
                                    src_v.at[q], isem).wait()
              pltpu.make_async_copy(dst_i.at[wid, pl.ds(0, IB)],
                                    dst_v.at[q], isem).wait()
              pltpu.async_copy(table.at[src_v.at[q, 0]], gbufs.at[0], gsem)
          pltpu.make_async_copy(table.at[src_v.at[p, jj]],
                                gbufs.at[jj % 2], gsem).wait()
          pltpu.sync_copy(gbufs.at[jj % 2], acc.at[dst_v.at[p, jj]], add=True)
        return c
      lax.fori_loop(0, nblk, block, 0)

    @pl.when(cid == 1)
    def _():
      def cstep(o, c):
        pltpu.sync_copy(dst_i.at[wid, pl.ds(o * IB, IB)], dst_v.at[0])
        for jj in range(IB):
          pltpu.sync_copy(gbufs.at[0], acc.at[dst_v.at[0, jj]], add=True)
        return c
      lax.fori_loop(0, nblk, cstep, 0)

    plsc.subcore_barrier()

    def wb(i, c):
      pltpu.sync_copy(acc.at[pl.ds(sid * RPT + i * CH, CH)], gbufs.at[1])
      pltpu.sync_copy(gbufs.at[1],
                      out.at[pl.ds(cid * NPAD + sid * RPT + i * CH, CH)])
      return c
    lax.fori_loop(0, RPT // CH, wb, 0)

  return pl.kernel(body, out_type=out_type, mesh=mesh, scratch_types=scratch)


def _make_sc_count(nch):
  mesh = plsc.VectorSubcoreMesh(core_axis_name="c", subcore_axis_name="s")
  out_type = jax.ShapeDtypeStruct((NC * NPAD, W), jnp.float32)
  scratch = [
      pltpu.VMEM((2, IB, CH), jnp.int32),
      pltpu.VMEM((CH, W), jnp.float32),
      pltpu.VMEM((16, W), jnp.float32),
      pltpu.VMEM_SHARED((NPAD, W), jnp.float32),
      pltpu.SemaphoreType.DMA,
  ]

  def body(dst_i, cnt, dst_v, obuf, zbuf, acc, isem):
    cid = lax.axis_index("c")
    sid = lax.axis_index("s")
    wid = cid * NS + sid
    z16 = jnp.zeros((16,), jnp.float32)
    ones16 = jnp.full((16,), 1.0, jnp.float32)
    for r in range(16):
      for k in range(W // 16):
        zbuf[r, pl.ds(k * 16, 16)] = z16

    def zero_acc(i, c):
      pltpu.sync_copy(zbuf, acc.at[pl.ds(sid * RPT + i * 16, 16)])
      return c
    lax.fori_loop(0, RPT // 16, zero_acc, 0)

    for r in range(CH):
      for k in range(W // 16):
        obuf[r, pl.ds(k * 16, 16)] = ones16

    plsc.subcore_barrier()

    pltpu.sync_copy(dst_i.at[wid, pl.ds(0, IB)], dst_v.at[0])

    def step(o, c):
      p = lax.rem(o, 2)
      q = lax.rem(o + 1, 2)

      @pl.when(o < nch // IB - 1)
      def _():
        pltpu.async_copy(dst_i.at[wid, pl.ds((o + 1) * IB, IB)],
                         dst_v.at[q], isem)
      for jj in range(IB):
        pltpu.sync_copy(obuf, acc.at[dst_v.at[p, jj]], add=True)

      @pl.when(o < nch // IB - 1)
      def _():
        pltpu.make_async_copy(dst_i.at[wid, pl.ds(0, IB)],
                              dst_v.at[q], isem).wait()
      return c
    lax.fori_loop(0, nch // IB, step, 0)

    plsc.subcore_barrier()

    def wb(i, c):
      pltpu.sync_copy(acc.at[pl.ds(sid * RPT + i * CH, CH)], obuf)
      pltpu.sync_copy(obuf, cnt.at[pl.ds(cid * NPAD + sid * RPT + i * CH, CH)])
      return c
    lax.fori_loop(0, RPT // CH, wb, 0)

  return pl.kernel(body, out_type=out_type, mesh=mesh, scratch_types=scratch)


def _bdot(a, b):
  return jnp.dot(a.astype(jnp.bfloat16), b.astype(jnp.bfloat16),
                 preferred_element_type=jnp.float32)


def _enc_body(x_r, w1_r, b1_r, g_r, bb_r, w2_r, b2_r, o_r):
  h = _bdot(x_r[...], w1_r[...]) + b1_r[...]
  m = jnp.mean(h, axis=-1, keepdims=True)
  c = h - m
  v = jnp.mean(c * c, axis=-1, keepdims=True)
  h = c * lax.rsqrt(v + 1e-5) * g_r[...] + bb_r[...]
  h = jnp.maximum(h, 0.0)
  o_r[...] = _bdot(h, w2_r[...]) + b2_r[...]


def _l1_body(s_r, cnt_r, h0_r, wl_r, bl_r, wr_r, a_r, o_r):
  inv = 1.0 / jnp.maximum(cnt_r[0][:, 0:1] + cnt_r[1][:, 0:1], 1.0)
  agg = (s_r[0] + s_r[1]) * inv
  t = _bdot(agg, wl_r[...]) + _bdot(h0_r[...], wr_r[...]) + bl_r[...]
  h1 = jnp.where(t >= 0, t, a_r[...] * t)
  o_r[0] = h1[:, :128]
  o_r[1] = h1[:, 128:]


def _l2_body(s_r, cnt_r, h1_r, wl_r, bl_r, wr_r, a_r, wl3_r, wr3_r, y_r, z_r):
  inv = 1.0 / jnp.maximum(cnt_r[0][:, 0:1] + cnt_r[1][:, 0:1], 1.0)
  t = (_bdot(s_r[0] * inv, wl_r[0:128])
       + _bdot(s_r[1] * inv, wl_r[128:256])
       + _bdot(h1_r[0], wr_r[0:128])
       + _bdot(h1_r[1], wr_r[128:256])
       + bl_r[...])
  h2 = jnp.where(t >= 0, t, a_r[...] * t)
  h2b = h2.astype(jnp.bfloat16)
  y_r[...] = jnp.dot(h2b, wl3_r[...].astype(jnp.bfloat16),
                     preferred_element_type=jnp.float32)
  z_r[...] = jnp.dot(h2b, wr3_r[...].astype(jnp.bfloat16),
                     preferred_element_type=jnp.float32)


def _fin_body(s_r, cnt_r, z_r, bl_r, a_r, o_r):
  inv = 1.0 / jnp.maximum(cnt_r[0][:, 0:1] + cnt_r[1][:, 0:1], 1.0)
  t = (s_r[0] + s_r[1]) * inv + z_r[...] + bl_r[...]
  o_r[...] = jnp.where(t >= 0, t, a_r[...] * t)


def _full(shape):
  return pl.BlockSpec(shape, lambda i: (0,) * len(shape))


def _pad_src(v, tot):
  pad = jnp.arange(tot - E, dtype=jnp.int32) % N
  return jnp.concatenate([v, pad])


def _pad_dst(v, tot):
  return jnp.concatenate([v, jnp.full((tot - E,), NPAD - 1, jnp.int32)])


def _pad_edges_split(v, fill, nch0, nch1):
  nchm = max(nch0, nch1)
  cap0 = NS * nch0 * CH
  s0, s1 = v[:cap0], v[cap0:]
  a0 = s0.reshape(NS, nch0, CH)
  a0 = jnp.concatenate(
      [a0, jnp.full((NS, nchm - nch0, CH), fill, jnp.int32)], axis=1)
  pad1 = NS * nch1 * CH - (E - cap0)
  a1 = jnp.concatenate([s1, jnp.full((pad1,), fill, jnp.int32)])
  a1 = a1.reshape(NS, nch1, CH)
  a1 = jnp.concatenate(
      [a1, jnp.full((NS, nchm - nch1, CH), fill, jnp.int32)], axis=1)
  return jnp.concatenate([a0, a1], axis=0)


def kernel(x, edge_index, batch_size, enc_w1, enc_b1, ln_g, ln_b, enc_w2,
           enc_b2, wl1, bl1, wr1, wl2, bl2, wr2, wl3, bl3, wr3, a1, a2, a3):
  f32 = jnp.float32
  xpad = jnp.zeros((NPAD, x.shape[1]), f32).at[:N].set(x)
  src = edge_index[0].astype(jnp.int32)
  dst = edge_index[1].astype(jnp.int32)

  src_a = _pad_src(src, NW * NCH_A * CH).reshape(NW, NCH_A, CH)
  dst_a = _pad_dst(dst, NW * NCH_A * CH).reshape(NW, NCH_A, CH)
  src_h = _pad_src(src, NS * NCH_B * CH).reshape(NS, NCH_B, CH)
  src_b = jnp.concatenate([src_h, src_h + NPAD], axis=0)
  dst_h = _pad_dst(dst, NS * NCH_B * CH).reshape(NS, NCH_B, CH)
  dst_b = jnp.concatenate([dst_h, dst_h], axis=0)

  enc = pl.pallas_call(
      _enc_body,
      grid=(GRID,),
      in_specs=[
          pl.BlockSpec((RBLK, 128), lambda i: (i, 0)),
          _full((128, 256)), _full((1, 256)), _full((1, 256)),
          _full((1, 256)), _full((256, 128)), _full((1, 128)),
      ],
      out_specs=pl.BlockSpec((RBLK, 128), lambda i: (i, 0)),
      out_shape=jax.ShapeDtypeStruct((NPAD, 128), f32),
  )
  h0 = enc(xpad, enc_w1, enc_b1.reshape(1, -1), ln_g.reshape(1, -1),
           ln_b.reshape(1, -1), enc_w2, enc_b2.reshape(1, -1))

  (sums0,) = _make_sc_sum(NCH_A, NCH_A)(h0, src_a, dst_a)
  cntm = _make_sc_count(NCH_A)(dst_a).reshape(2, NPAD, 128)

  l1 = pl.pallas_call(
      _l1_body,
      grid=(GRID,),
      in_specs=[
          pl.BlockSpec((2, RBLK, 128), lambda i: (0, i, 0)),
          pl.BlockSpec((2, RBLK, 128), lambda i: (0, i, 0)),
          pl.BlockSpec((RBLK, 128), lambda i: (i, 0)),
          _full((128, 256)), _full((1, 256)), _full((128, 256)),
          _full((1, 256)),
      ],
      out_specs=pl.BlockSpec((2, RBLK, 128), lambda i: (0, i, 0)),
      out_shape=jax.ShapeDtypeStruct((2, NPAD, 128), f32),
  )
  h1c = l1(sums0.reshape(2, NPAD, 128), cntm, h0,
           wl1, bl1.reshape(1, -1), wr1, a1.reshape(1, -1))

  (sums1,) = _make_sc_sum(NCH_B, NCH_B)(h1c.reshape(NC * NPAD, 128), src_b, dst_b)

  l2 = pl.pallas_call(
      _l2_body,
      grid=(GRID,),
      in_specs=[
          pl.BlockSpec((2, RBLK, 128), lambda i: (0, i, 0)),
          pl.BlockSpec((2, RBLK, 128), lambda i: (0, i, 0)),
          pl.BlockSpec((2, RBLK, 128), lambda i: (0, i, 0)),
          _full((256, 512)), _full((1, 512)), _full((256, 512)),
          _full((1, 512)), _full((512, 128)), _full((512, 128)),
      ],
      out_specs=[
          pl.BlockSpec((RBLK, 128), lambda i: (i, 0)),
          pl.BlockSpec((RBLK, 128), lambda i: (i, 0)),
      ],
      out_shape=[
          jax.ShapeDtypeStruct((NPAD, 128), f32),
          jax.ShapeDtypeStruct((NPAD, 128), f32),
      ],
  )
  y, z = l2(sums1.reshape(2, NPAD, 128), cntm, h1c,
            wl2, bl2.reshape(1, -1), wr2, a2.reshape(1, -1), wl3, wr3)

  (sums2,) = _make_sc_sum(NCH_A, NCH_A)(y, src_a, dst_a)

  fin = pl.pallas_call(
      _fin_body,
      grid=(GRID,),
      in_specs=[
          pl.BlockSpec((2, RBLK, 128), lambda i: (0, i, 0)),
          pl.BlockSpec((2, RBLK, 128), lambda i: (0, i, 0)),
          pl.BlockSpec((RBLK, 128), lambda i: (i, 0)),
          _full((1, 128)), _full((1, 128)),
      ],
      out_specs=pl.BlockSpec((RBLK, 128), lambda i: (i, 0)),
      out_shape=jax.ShapeDtypeStruct((NPAD, 128), f32),
  )
  o = fin(sums2.reshape(2, NPAD, 128), cntm, z,
          bl3.reshape(1, -1), a3.reshape(1, -1))

  return lax.dynamic_slice_in_dim(o[:N], batch_size - 1024, 1024, axis=0)


# speedup vs baseline: 1.3485x; 1.0141x over previous
import jax
import jax.numpy as jnp
from jax import lax
from jax.experimental import pallas as pl
from jax.experimental.pallas import tpu as pltpu
from jax.experimental.pallas import tpu_sc as plsc

N = 10000
NPAD = 10240
E = 320000
NS = 16
NC = 2
NW = NC * NS
CH = 128
W = 128
IB = 8
NCH_A = 80
NCH_B = 160
RPT = NPAD // NS
RBLK = 2048
GRID = NPAD // RBLK


def _make_sc_sum(nch0, nch1):
  nchm = max(nch0, nch1)
  mesh = plsc.VectorSubcoreMesh(core_axis_name="c", subcore_axis_name="s")
  out_type = [jax.ShapeDtypeStruct((NC * NPAD, W), jnp.float32)]
  scratch = [
      pltpu.VMEM((2, IB, CH), jnp.int32),
      pltpu.VMEM((2, IB, CH), jnp.int32),
      pltpu.VMEM((2, CH, W), jnp.float32),
      pltpu.VMEM((16, W), jnp.float32),
      pltpu.VMEM_SHARED((NPAD, W), jnp.float32),
      pltpu.SemaphoreType.DMA,
      pltpu.SemaphoreType.DMA,
  ]
  def body(table, src_i, dst_i, sums, src_v, dst_v, gbufs, zbuf, acc,
           gsem, isem):
    cid = lax.axis_index("c")
    sid = lax.axis_index("s")
    wid = cid * NS + sid
    nblk = jnp.where(cid == 0, nch0 // IB, nch1 // IB)
    z16 = jnp.zeros((16,), jnp.float32)
    for r in range(16):
      for k in range(W // 16):
        zbuf[r, pl.ds(k * 16, 16)] = z16

    def zero_acc(i, c):
      pltpu.sync_copy(zbuf, acc.at[pl.ds(sid * RPT + i * 16, 16)])
      return c
    lax.fori_loop(0, RPT // 16, zero_acc, 0)

    plsc.subcore_barrier()

    pltpu.sync_copy(src_i.at[wid, pl.ds(0, IB)], src_v.at[0])
    pltpu.sync_copy(dst_i.at[wid, pl.ds(0, IB)], dst_v.at[0])
    pltpu.async_copy(table.at[src_v.at[0, 0]], gbufs.at[0], gsem)

    def block(o, c):
      p = lax.rem(o, 2)
      q = lax.rem(o + 1, 2)
      not_last = o < nblk - 1

      @pl.when(not_last)
      def _():
        pltpu.async_copy(src_i.at[wid, pl.ds((o + 1) * IB, IB)],
                         src_v.at[q], isem)
        pltpu.async_copy(dst_i.at[wid, pl.ds((o + 1) * IB, IB)],
                         dst_v.at[q], isem)
      for jj in range(IB):
        if jj + 1 < IB:
          pltpu.async_copy(table.at[src_v.at[p, jj + 1]],
                           gbufs.at[(jj + 1) % 2], gsem)
        else:
          @pl.when(not_last)
          def _():
            pltpu.make_async_copy(src_i.at[wid, pl.ds(0, IB)],
                                  src_v.at[q], isem).wait()
            pltpu.make_async_copy(dst_i.at[wid, pl.ds(0, IB)],
                                  dst_v.at[q], isem).wait()
            pltpu.async_copy(table.at[src_v.at[q, 0]], gbufs.at[0], gsem)
        pltpu.make_async_copy(table.at[src_v.at[p, jj]],
                              gbufs.at[jj % 2], gsem).wait()
        pltpu.sync_copy(gbufs.at[jj % 2], acc.at[dst_v.at[p, jj]], add=True)
      return c
    lax.fori_loop(0, nblk, block, 0)

    plsc.subcore_barrier()

    def wb(i, c):
      pltpu.sync_copy(acc.at[pl.ds(sid * RPT + i * CH, CH)], gbufs.at[0])
      pltpu.sync_copy(gbufs.at[0],
                      sums.at[pl.ds(cid * NPAD + sid * RPT + i * CH, CH)])
      return c
    lax.fori_loop(0, RPT // CH, wb, 0)

  return pl.kernel(body, out_type=tuple(out_type), mesh=mesh,
                   scratch_types=scratch)


def _make_sc_gather_count(nch):
  nblk = nch // IB
  mesh = plsc.VectorSubcoreMesh(core_axis_name="c", subcore_axis_name="s")
  out_type = jax.ShapeDtypeStruct((NC * NPAD, W), jnp.float32)
  scratch = [
      pltpu.VMEM((2, IB, CH), jnp.int32),
      pltpu.VMEM((2, IB, CH), jnp.int32),
      pltpu.VMEM((2, CH, W), jnp.float32),
      pltpu.VMEM((16, W), jnp.float32),
      pltpu.VMEM_SHARED((NPAD, W), jnp.float32),
      pltpu.SemaphoreType.DMA,
      pltpu.SemaphoreType.DMA,
  ]

  def body(table, src_i, dst_i, out, src_v, dst_v, gbufs, zbuf, acc,
           gsem, isem):
    cid = lax.axis_index("c")
    sid = lax.axis_index("s")
    wid = cid * NS + sid
    z16 = jnp.zeros((16,), jnp.float32)
    ones16 = jnp.full((16,), 1.0, jnp.float32)
    for r in range(16):
      for k in range(W // 16):
        zbuf[r, pl.ds(k * 16, 16)] = z16

    def zero_acc(i, c):
      pltpu.sync_copy(zbuf, acc.at[pl.ds(sid * RPT + i * 16, 16)])
      return c
    lax.fori_loop(0, RPT // 16, zero_acc, 0)

    @pl.when(cid == 1)
    def _():
      for r in range(CH):
        for k in range(W // 16):
          gbufs[0, r, pl.ds(k * 16, 16)] = ones16

    plsc.subcore_barrier()

    @pl.when(cid == 0)
    def _():
      pltpu.sync_copy(src_i.at[wid, pl.ds(0, IB)], src_v.at[0])
      pltpu.sync_copy(dst_i.at[wid, pl.ds(0, IB)], dst_v.at[0])
      pltpu.async_copy(table.at[src_v.at[0, 0]], gbufs.at[0], gsem)

      def block(o, c):
        p = lax.rem(o, 2)
        q = lax.rem(o + 1, 2)
        not_last = o < nblk - 1

        @pl.when(not_last)
        def _():
          pltpu.async_copy(src_i.at[wid, pl.ds((o + 1) * IB, IB)],
                           src_v.at[q], isem)
          pltpu.async_copy(dst_i.at[wid, pl.ds((o + 1) * IB, IB)],
                           dst_v.at[q], isem)
        for jj in range(IB):
          if jj + 1 < IB:
            pltpu.async_copy(table.at[src_v.at[p, jj + 1]],
                             gbufs.at[(jj + 1) % 2], gsem)
          else:
            @pl.when(not_last)
            def _():
              pltpu.make_async_copy(src_i.at[wid, pl.ds(0, IB)],
                                    src_v.at[q], isem).wait()
              pltpu.make_async_copy(dst_i.at[wid, pl.ds(0, IB)],
                                    dst_v.at[q], isem).wait()
              pltpu.async_copy(table.at[src_v.at[q, 0]], gbufs.at[0], gsem)
          pltpu.make_async_copy(table.at[src_v.at[p, jj]],
                                gbufs.at[jj % 2], gsem).wait()
          pltpu.sync_copy(gbufs.at[jj % 2], acc.at[dst_v.at[p, jj]], add=True)
        return c
      lax.fori_loop(0, nblk, block, 0)

    @pl.when(cid == 1)
    def _():
      def cstep(o, c):
        pltpu.sync_copy(dst_i.at[wid, pl.ds(o * IB, IB)], dst_v.at[0])
        for jj in range(IB):
          pltpu.sync_copy(gbufs.at[0], acc.at[dst_v.at[0, jj]], add=True)
        return c
      lax.fori_loop(0, nblk, cstep, 0)

    plsc.subcore_barrier()

    def wb(i, c):
      pltpu.sync_copy(acc.at[pl.ds(sid * RPT + i * CH, CH)], gbufs.at[1])
      pltpu.sync_copy(gbufs.at[1],
                      out.at[pl.ds(cid * NPAD + sid * RPT + i * CH, CH)])
      return c
    lax.fori_loop(0, RPT // CH, wb, 0)

  return pl.kernel(body, out_type=out_type, mesh=mesh, scratch_types=scratch)


def _make_sc_count(nch):
  mesh = plsc.VectorSubcoreMesh(core_axis_name="c", subcore_axis_name="s")
  out_type = jax.ShapeDtypeStruct((NC * NPAD, W), jnp.float32)
  scratch = [
      pltpu.VMEM((2, IB, CH), jnp.int32),
      pltpu.VMEM((CH, W), jnp.float32),
      pltpu.VMEM((16, W), jnp.float32),
      pltpu.VMEM_SHARED((NPAD, W), jnp.float32),
      pltpu.SemaphoreType.DMA,
  ]

  def body(dst_i, cnt, dst_v, obuf, zbuf, acc, isem):
    cid = lax.axis_index("c")
    sid = lax.axis_index("s")
    wid = cid * NS + sid
    z16 = jnp.zeros((16,), jnp.float32)
    ones16 = jnp.full((16,), 1.0, jnp.float32)
    for r in range(16):
      for k in range(W // 16):
        zbuf[r, pl.ds(k * 16, 16)] = z16

    def zero_acc(i, c):
      pltpu.sync_copy(zbuf, acc.at[pl.ds(sid * RPT + i * 16, 16)])
      return c
    lax.fori_loop(0, RPT // 16, zero_acc, 0)

    for r in range(CH):
      for k in range(W // 16):
        obuf[r, pl.ds(k * 16, 16)] = ones16

    plsc.subcore_barrier()

    pltpu.sync_copy(dst_i.at[wid, pl.ds(0, IB)], dst_v.at[0])

    def step(o, c):
      p = lax.rem(o, 2)
      q = lax.rem(o + 1, 2)

      @pl.when(o < nch // IB - 1)
      def _():
        pltpu.async_copy(dst_i.at[wid, pl.ds((o + 1) * IB, IB)],
                         dst_v.at[q], isem)
      for jj in range(IB):
        pltpu.sync_copy(obuf, acc.at[dst_v.at[p, jj]], add=True)

      @pl.when(o < nch // IB - 1)
      def _():
        pltpu.make_async_copy(dst_i.at[wid, pl.ds(0, IB)],
                              dst_v.at[q], isem).wait()
      return c
    lax.fori_loop(0, nch // IB, step, 0)

    plsc.subcore_barrier()

    def wb(i, c):
      pltpu.sync_copy(acc.at[pl.ds(sid * RPT + i * CH, CH)], obuf)
      pltpu.sync_copy(obuf, cnt.at[pl.ds(cid * NPAD + sid * RPT + i * CH, CH)])
      return c
    lax.fori_loop(0, RPT // CH, wb, 0)

  return pl.kernel(body, out_type=out_type, mesh=mesh, scratch_types=scratch)


def _bdot(a, b):
  return jnp.dot(a.astype(jnp.bfloat16), b.astype(jnp.bfloat16),
                 preferred_element_type=jnp.float32)


def _enc_body(x_r, w1_r, b1_r, g_r, bb_r, w2_r, b2_r, o_r):
  h = _bdot(x_r[...], w1_r[...]) + b1_r[...]
  m = jnp.mean(h, axis=-1, keepdims=True)
  c = h - m
  v = jnp.mean(c * c, axis=-1, keepdims=True)
  h = c * lax.rsqrt(v + 1e-5) * g_r[...] + bb_r[...]
  h = jnp.maximum(h, 0.0)
  o_r[...] = _bdot(h, w2_r[...]) + b2_r[...]


def _l1_body(s_r, cnt_r, h0_r, wl_r, bl_r, wr_r, a_r, o_r):
  inv = 1.0 / jnp.maximum(cnt_r[0][:, 0:1] + cnt_r[1][:, 0:1], 1.0)
  agg = (s_r[0] + s_r[1]) * inv
  t = _bdot(agg, wl_r[...]) + _bdot(h0_r[...], wr_r[...]) + bl_r[...]
  h1 = jnp.where(t >= 0, t, a_r[...] * t)
  o_r[0] = h1[:, :128]
  o_r[1] = h1[:, 128:]


def _l2_body(s_r, cnt_r, h1_r, wl_r, bl_r, wr_r, a_r, wl3_r, wr3_r, y_r, z_r):
  inv = 1.0 / jnp.maximum(cnt_r[0][:, 0:1] + cnt_r[1][:, 0:1], 1.0)
  t = (_bdot(s_r[0] * inv, wl_r[0:128])
       + _bdot(s_r[1] * inv, wl_r[128:256])
       + _bdot(h1_r[0], wr_r[0:128])
       + _bdot(h1_r[1], wr_r[128:256])
       + bl_r[...])
  h2 = jnp.where(t >= 0, t, a_r[...] * t)
  h2b = h2.astype(jnp.bfloat16)
  y_r[...] = jnp.dot(h2b, wl3_r[...].astype(jnp.bfloat16),
                     preferred_element_type=jnp.float32)
  z_r[...] = jnp.dot(h2b, wr3_r[...].astype(jnp.bfloat16),
                     preferred_element_type=jnp.float32)


def _fin_body(s_r, cnt_r, z_r, bl_r, a_r, o_r):
  inv = 1.0 / jnp.maximum(cnt_r[0][:, 0:1] + cnt_r[1][:, 0:1], 1.0)
  t = (s_r[0] + s_r[1]) * inv + z_r[...] + bl_r[...]
  o_r[...] = jnp.where(t >= 0, t, a_r[...] * t)


def _full(shape):
  return pl.BlockSpec(shape, lambda i: (0,) * len(shape))


def _pad_src(v, tot):
  pad = jnp.arange(tot - E, dtype=jnp.int32) % N
  return jnp.concatenate([v, pad])


def _pad_dst(v, tot):
  return jnp.concatenate([v, jnp.full((tot - E,), NPAD - 1, jnp.int32)])


def _pad_edges_split(v, fill, nch0, nch1):
  nchm = max(nch0, nch1)
  cap0 = NS * nch0 * CH
  s0, s1 = v[:cap0], v[cap0:]
  a0 = s0.reshape(NS, nch0, CH)
  a0 = jnp.concatenate(
      [a0, jnp.full((NS, nchm - nch0, CH), fill, jnp.int32)], axis=1)
  pad1 = NS * nch1 * CH - (E - cap0)
  a1 = jnp.concatenate([s1, jnp.full((pad1,), fill, jnp.int32)])
  a1 = a1.reshape(NS, nch1, CH)
  a1 = jnp.concatenate(
      [a1, jnp.full((NS, nchm - nch1, CH), fill, jnp.int32)], axis=1)
  return jnp.concatenate([a0, a1], axis=0)


def kernel(x, edge_index, batch_size, enc_w1, enc_b1, ln_g, ln_b, enc_w2,
           enc_b2, wl1, bl1, wr1, wl2, bl2, wr2, wl3, bl3, wr3, a1, a2, a3):
  f32 = jnp.float32
  xpad = jnp.zeros((NPAD, x.shape[1]), f32).at[:N].set(x)
  src = edge_index[0].astype(jnp.int32)
  dst = edge_index[1].astype(jnp.int32)

  src_a = _pad_src(src, NW * NCH_A * CH).reshape(NW, NCH_A, CH)
  dst_a = _pad_dst(dst, NW * NCH_A * CH).reshape(NW, NCH_A, CH)
  src_h = _pad_src(src, NS * NCH_B * CH).reshape(NS, NCH_B, CH)
  src_b = jnp.concatenate([src_h, src_h + NPAD], axis=0)
  dst_h = _pad_dst(dst, NS * NCH_B * CH).reshape(NS, NCH_B, CH)
  dst_b = jnp.concatenate([dst_h, dst_h], axis=0)

  enc = pl.pallas_call(
      _enc_body,
      grid=(GRID,),
      in_specs=[
          pl.BlockSpec((RBLK, 128), lambda i: (i, 0)),
          _full((128, 256)), _full((1, 256)), _full((1, 256)),
          _full((1, 256)), _full((256, 128)), _full((1, 128)),
      ],
      out_specs=pl.BlockSpec((RBLK, 128), lambda i: (i, 0)),
      out_shape=jax.ShapeDtypeStruct((NPAD, 128), f32),
  )
  h0 = enc(xpad, enc_w1, enc_b1.reshape(1, -1), ln_g.reshape(1, -1),
           ln_b.reshape(1, -1), enc_w2, enc_b2.reshape(1, -1))

  (sums0,) = _make_sc_sum(NCH_A, NCH_A)(h0, src_a, dst_a)
  cntm = _make_sc_count(NCH_A)(dst_a).reshape(2, NPAD, 128)

  l1 = pl.pallas_call(
      _l1_body,
      grid=(GRID,),
      in_specs=[
          pl.BlockSpec((2, RBLK, 128), lambda i: (0, i, 0)),
          pl.BlockSpec((2, RBLK, 128), lambda i: (0, i, 0)),
          pl.BlockSpec((RBLK, 128), lambda i: (i, 0)),
          _full((128, 256)), _full((1, 256)), _full((128, 256)),
          _full((1, 256)),
      ],
      out_specs=pl.BlockSpec((2, RBLK, 128), lambda i: (0, i, 0)),
      out_shape=jax.ShapeDtypeStruct((2, NPAD, 128), f32),
  )
  h1c = l1(sums0.reshape(2, NPAD, 128), cntm, h0,
           wl1, bl1.reshape(1, -1), wr1, a1.reshape(1, -1))

  (sums1,) = _make_sc_sum(NCH_B, NCH_B)(h1c.reshape(NC * NPAD, 128), src_b, dst_b)

  l2 = pl.pallas_call(
      _l2_body,
      grid=(GRID,),
      in_specs=[
          pl.BlockSpec((2, RBLK, 128), lambda i: (0, i, 0)),
          pl.BlockSpec((2, RBLK, 128), lambda i: (0, i, 0)),
          pl.BlockSpec((2, RBLK, 128), lambda i: (0, i, 0)),
          _full((256, 512)), _full((1, 512)), _full((256, 512)),
          _full((1, 512)), _full((512, 128)), _full((512, 128)),
      ],
      out_specs=[
          pl.BlockSpec((RBLK, 128), lambda i: (i, 0)),
          pl.BlockSpec((RBLK, 128), lambda i: (i, 0)),
      ],
      out_shape=[
          jax.ShapeDtypeStruct((NPAD, 128), f32),
          jax.ShapeDtypeStruct((NPAD, 128), f32),
      ],
  )
  y, z = l2(sums1.reshape(2, NPAD, 128), cntm, h1c,
            wl2, bl2.reshape(1, -1), wr2, a2.reshape(1, -1), wl3, wr3)

  (sums2,) = _make_sc_sum(NCH_A, NCH_A)(y, src_a, dst_a)

  fin = pl.pallas_call(
      _fin_body,
      grid=(GRID,),
      in_specs=[
          pl.BlockSpec((2, RBLK, 128), lambda i: (0, i, 0)),
          pl.BlockSpec((2, RBLK, 128), lambda i: (0, i, 0)),
          pl.BlockSpec((RBLK, 128), lambda i: (i, 0)),
          _full((1, 128)), _full((1, 128)),
      ],
      out_specs=pl.BlockSpec((RBLK, 128), lambda i: (i, 0)),
      out_shape=jax.ShapeDtypeStruct((NPAD, 128), f32),
  )
  o = fin(sums2.reshape(2, NPAD, 128), cntm, z,
          bl3.reshape(1, -1), a3.reshape(1, -1))

  return lax.dynamic_slice_in_dim(o[:N], batch_size - 1024, 1024, axis=0)
